# Initial kernel scaffold; baseline (speedup 1.0000x reference)
import jax
import jax.numpy as jnp
from jax.experimental import pallas as pl


def kernel(x, edge_index, edge_features, batch_size, W1, a_src1, a_dst1, We1, ae1, b1, W2, a_src2, a_dst2, We2, ae2, b2):
    raise NotImplementedError("write your pallas kernel here")


# speedup vs baseline: 3.6156x; 3.6156x over previous
import functools

import jax
import jax.numpy as jnp
from jax import lax
from jax.experimental import pallas as pl
from jax.experimental.pallas import tpu as pltpu
from jax.experimental.pallas import tpu_sc as plsc

NC = 2
NS = 16
NW = NC * NS
L = 16
C = 128


def _decode(x, src, dst):
    n, d = x.shape
    e = src.shape[0]
    half = e // 2
    nchunk = half // C
    nt = (nchunk + NW - 1) // NW

    mesh = plsc.VectorSubcoreMesh(
        core_axis_name="c", subcore_axis_name="s",
        num_cores=NC, num_subcores=NS)

    @functools.partial(
        pl.kernel,
        out_type=jax.ShapeDtypeStruct((half,), jnp.float32),
        mesh=mesh,
        scratch_types=[
            pltpu.VMEM((C,), jnp.int32),
            pltpu.VMEM((C,), jnp.int32),
            pltpu.VMEM((C,), jnp.int32),
            pltpu.VMEM((C,), jnp.int32),
            pltpu.VMEM((C, d), jnp.float32),
            pltpu.VMEM((C, d), jnp.float32),
            pltpu.VMEM((C, d), jnp.float32),
            pltpu.VMEM((C, d), jnp.float32),
            pltpu.VMEM((C,), jnp.float32),
            pltpu.SemaphoreType.DMA,
        ],
        compiler_params=pltpu.CompilerParams(needs_layout_passes=False),
    )
    def decode(x_hbm, src_hbm, dst_hbm, out_hbm,
               is1, id1, is2, id2, rs1, rd1, rs2, rd2, ob, sem):
        wid = lax.axis_index("s") * NC + lax.axis_index("c")
        lane = lax.iota(jnp.int32, L)

        @pl.loop(0, nt)
        def _t(t):
            cid = t * NW + wid

            @pl.when(cid < nchunk)
            def _():
                base = cid * C
                pltpu.sync_copy(src_hbm.at[pl.ds(base, C)], is1)
                pltpu.sync_copy(dst_hbm.at[pl.ds(base, C)], id1)
                pltpu.sync_copy(src_hbm.at[pl.ds(base + half, C)], is2)
                pltpu.sync_copy(dst_hbm.at[pl.ds(base + half, C)], id2)
                c1 = pltpu.async_copy(x_hbm.at[is1], rs1, sem)
                c2 = pltpu.async_copy(x_hbm.at[id1], rd1, sem)
                c3 = pltpu.async_copy(x_hbm.at[is2], rs2, sem)
                c4 = pltpu.async_copy(x_hbm.at[id2], rd2, sem)
                c1.wait()
                c2.wait()
                c3.wait()
                c4.wait()
                @pl.loop(0, C)
                def _j(j):
                    acc = jnp.zeros((L,), jnp.float32)
                    for c0 in range(d // L):
                        sl = pl.ds(c0 * L, L)
                        acc = acc + rs1[j, sl] * rd1[j, sl]
                        acc = acc + rs2[j, sl] * rd2[j, sl]
                    r = jnp.sum(acc)
                    plsc.store_scatter(
                        ob, [jnp.full((L,), 0, jnp.int32) + j],
                        jnp.full((L,), 0, jnp.float32) + r,
                        mask=lane == 0)
                pltpu.sync_copy(ob, out_hbm.at[pl.ds(base, C)])

    return decode(x, src, dst)


def kernel(x, edge_index, edge_features, batch_size,
           W1, a_src1, a_dst1, We1, ae1, b1,
           W2, a_src2, a_dst2, We2, ae2, b2):
    res_half = _decode(x, edge_index[0], edge_index[1])
    return res_half.reshape((1000, -1))


# speedup vs baseline: 6.6542x; 1.8404x over previous
import functools

import jax
import jax.numpy as jnp
from jax import lax
from jax.experimental import pallas as pl
from jax.experimental.pallas import tpu as pltpu
from jax.experimental.pallas import tpu_sc as plsc

NC = 2
NS = 16
NW = NC * NS
L = 16
C = 40


def _decode(x, src, dst):
    n, d = x.shape
    e = src.shape[0]
    half = e // 2
    per_w = half // NW
    nk = per_w // C

    mesh = plsc.VectorSubcoreMesh(
        core_axis_name="c", subcore_axis_name="s",
        num_cores=NC, num_subcores=NS)

    @functools.partial(
        pl.kernel,
        out_type=jax.ShapeDtypeStruct((half,), jnp.float32),
        mesh=mesh,
        scratch_types=[
            pltpu.VMEM((per_w,), jnp.int32),
            pltpu.VMEM((per_w,), jnp.int32),
            pltpu.VMEM((per_w,), jnp.int32),
            pltpu.VMEM((per_w,), jnp.int32),
            [pltpu.VMEM((C, d), jnp.float32) for _ in range(4)],
            [pltpu.VMEM((C, d), jnp.float32) for _ in range(4)],
            pltpu.VMEM((per_w,), jnp.float32),
            pltpu.SemaphoreType.DMA,
            pltpu.SemaphoreType.DMA,
        ],
        compiler_params=pltpu.CompilerParams(needs_layout_passes=False),
    )
    def decode(x_hbm, src_hbm, dst_hbm, out_hbm,
               i1, i2, i3, i4, set0, set1, ob, sem0, sem1):
        wid = lax.axis_index("s") * NC + lax.axis_index("c")
        base0 = wid * per_w
        lane = lax.iota(jnp.int32, L)
        idxs = (i1, i2, i3, i4)
        sets = (set0, set1)
        sems = (sem0, sem1)

        pltpu.sync_copy(src_hbm.at[pl.ds(base0, per_w)], i1)
        pltpu.sync_copy(dst_hbm.at[pl.ds(base0, per_w)], i2)
        pltpu.sync_copy(src_hbm.at[pl.ds(base0 + half, per_w)], i3)
        pltpu.sync_copy(dst_hbm.at[pl.ds(base0 + half, per_w)], i4)

        def fire(k, s):
            o = k * C
            for q in range(4):
                pltpu.async_copy(
                    x_hbm.at[idxs[q].at[pl.ds(o, C)]], sets[s][q], sems[s])

        def drain(s):
            for q in range(4):
                pltpu.make_async_copy(
                    x_hbm.at[idxs[q].at[pl.ds(0, C)]], sets[s][q],
                    sems[s]).wait()

        def compute(k, s):
            rs1, rd1, rs2, rd2 = sets[s]

            @pl.loop(0, C)
            def _j(j):
                acc = jnp.zeros((L,), jnp.float32)
                for c0 in range(d // L):
                    sl = pl.ds(c0 * L, L)
                    acc = acc + rs1[j, sl] * rd1[j, sl]
                    acc = acc + rs2[j, sl] * rd2[j, sl]
                r = jnp.sum(acc)
                plsc.store_scatter(
                    ob, [jnp.full((L,), 0, jnp.int32) + (k * C + j)],
                    jnp.full((L,), 0.0, jnp.float32) + r,
                    mask=lane == 0)

        fire(0, 0)

        @pl.loop(0, (nk - 1) // 2)
        def _t(tt):
            k = tt * 2
            fire(k + 1, 1)
            drain(0)
            compute(k, 0)
            fire(k + 2, 0)
            drain(1)
            compute(k + 1, 1)

        drain(0)
        compute(nk - 1, 0)
        pltpu.sync_copy(ob, out_hbm.at[pl.ds(base0, per_w)])

    return decode(x, src, dst)


def kernel(x, edge_index, edge_features, batch_size,
           W1, a_src1, a_dst1, We1, ae1, b1,
           W2, a_src2, a_dst2, We2, ae2, b2):
    res_half = _decode(x, edge_index[0], edge_index[1])
    return res_half.reshape((1000, -1))


# speedup vs baseline: 7.0303x; 1.0565x over previous
import functools

import jax
import jax.numpy as jnp
from jax import lax
from jax.experimental import pallas as pl
from jax.experimental.pallas import tpu as pltpu
from jax.experimental.pallas import tpu_sc as plsc

NC = 2
NS = 16
NW = NC * NS
L = 16
C = 40


def _decode(x, src, dst):
    n, d = x.shape
    e = src.shape[0]
    half = e // 2
    per_w = half // NW
    nk = per_w // C

    mesh = plsc.VectorSubcoreMesh(
        core_axis_name="c", subcore_axis_name="s",
        num_cores=NC, num_subcores=NS)

    @functools.partial(
        pl.kernel,
        out_type=jax.ShapeDtypeStruct((half,), jnp.float32),
        mesh=mesh,
        scratch_types=[
            [pltpu.VMEM((C,), jnp.int32) for _ in range(4)],
            [pltpu.VMEM((C,), jnp.int32) for _ in range(4)],
            [pltpu.VMEM((C, d), jnp.float32) for _ in range(4)],
            [pltpu.VMEM((C, d), jnp.float32) for _ in range(4)],
            pltpu.VMEM((per_w,), jnp.float32),
            pltpu.VMEM_SHARED((n, d), jnp.float32),
            pltpu.SemaphoreType.DMA,
            pltpu.SemaphoreType.DMA,
            pltpu.SemaphoreType.DMA,
            pltpu.SemaphoreType.DMA,
        ],
        compiler_params=pltpu.CompilerParams(needs_layout_passes=False),
    )
    def decode(x_hbm, src_hbm, dst_hbm, out_hbm,
               ia0, ia1, set0, set1, ob, xs, si0, si1, sr0, sr1):
        wid = lax.axis_index("s") * NC + lax.axis_index("c")
        sid = lax.axis_index("s")
        base0 = wid * per_w
        lane = lax.iota(jnp.int32, L)
        ias = (ia0, ia1)
        sets = (set0, set1)
        sis = (si0, si1)
        srs = (sr0, sr1)

        rps = (n // NS) // 8 * 8
        off = pl.multiple_of(sid * rps, 8)
        pltpu.sync_copy(x_hbm.at[pl.ds(off, rps)], xs.at[pl.ds(off, rps)])
        tail = n - rps * NS
        if tail:
            @pl.when(sid == 0)
            def _tail():
                pltpu.sync_copy(x_hbm.at[pl.ds(rps * NS, tail)],
                                xs.at[pl.ds(rps * NS, tail)])
        plsc.subcore_barrier()

        def fire_idx(k, p):
            o = base0 + k * C
            pltpu.async_copy(src_hbm.at[pl.ds(o, C)], ias[p][0], sis[p])
            pltpu.async_copy(dst_hbm.at[pl.ds(o, C)], ias[p][1], sis[p])
            pltpu.async_copy(src_hbm.at[pl.ds(o + half, C)], ias[p][2],
                             sis[p])
            pltpu.async_copy(dst_hbm.at[pl.ds(o + half, C)], ias[p][3],
                             sis[p])

        def drain_idx(p):
            for q in range(4):
                pltpu.make_async_copy(src_hbm.at[pl.ds(0, C)], ias[p][q],
                                      sis[p]).wait()

        def fire_rows(p):
            for q in range(4):
                pltpu.async_copy(xs.at[ias[p][q]], sets[p][q], srs[p])

        def drain_rows(p):
            for q in range(4):
                pltpu.make_async_copy(xs.at[ias[p][q]], sets[p][q],
                                      srs[p]).wait()

        def compute(k, p):
            rs1, rd1, rs2, rd2 = sets[p]

            @pl.loop(0, C)
            def _j(j):
                acc = jnp.zeros((L,), jnp.float32)
                for c0 in range(d // L):
                    sl = pl.ds(c0 * L, L)
                    acc = acc + rs1[j, sl] * rd1[j, sl]
                    acc = acc + rs2[j, sl] * rd2[j, sl]
                r = jnp.sum(acc)
                plsc.store_scatter(
                    ob, [jnp.full((L,), 0, jnp.int32) + (k * C + j)],
                    jnp.full((L,), 0.0, jnp.float32) + r,
                    mask=lane == 0)

        fire_idx(0, 0)
        fire_idx(1, 1)
        drain_idx(0)
        fire_rows(0)

        def step(k, p):
            q = 1 - p

            @pl.when(k + 1 < nk)
            def _():
                drain_idx(q)
                fire_rows(q)
            drain_rows(p)

            @pl.when(k + 2 < nk)
            def _():
                fire_idx(k + 2, p)
            compute(k, p)

        @pl.loop(0, nk // 2)
        def _t(tt):
            k = tt * 2
            step(k, 0)
            step(k + 1, 1)

        if nk % 2:
            step(nk - 1, 0)
        pltpu.sync_copy(ob, out_hbm.at[pl.ds(base0, per_w)])

    return decode(x, src, dst)


def kernel(x, edge_index, edge_features, batch_size,
           W1, a_src1, a_dst1, We1, ae1, b1,
           W2, a_src2, a_dst2, We2, ae2, b2):
    res_half = _decode(x, edge_index[0], edge_index[1])
    return res_half.reshape((1000, -1))
